# baseline (speedup 1.0000x reference)
import functools

import jax
import jax.numpy as jnp
from jax import lax
from jax.experimental import pallas as pl
from jax.experimental.pallas import tpu as pltpu
from jax.experimental.pallas import tpu_sc as plsc

EMBED = 64
B = 16384
NC = 2
NS = 16
NW = NC * NS
CHUNK = 128
B_PER_W = B // NW
NCHUNK = B_PER_W // CHUNK

_mesh = plsc.VectorSubcoreMesh(core_axis_name="c", subcore_axis_name="s")


@functools.partial(
    pl.kernel,
    mesh=_mesh,
    out_type=jax.ShapeDtypeStruct((NW, NCHUNK, CHUNK, EMBED), jnp.float32),
    scratch_types=[
        pltpu.VMEM((NCHUNK, CHUNK), jnp.int32),
        pltpu.VMEM((NCHUNK, CHUNK, EMBED), jnp.float32),
        pltpu.SemaphoreType.DMA,
    ],
    compiler_params=pltpu.CompilerParams(use_tc_tiling_on_sc=False),
)
def _sc_gather(idx_hbm, table_hbm, out_hbm, idx_v, rows_v, sem):
    wid = lax.axis_index("s") * NC + lax.axis_index("c")
    pltpu.sync_copy(idx_hbm.at[wid], idx_v)
    copies = [
        pltpu.async_copy(table_hbm.at[idx_v.at[j]], rows_v.at[j], sem)
        for j in range(NCHUNK)
    ]
    for c in copies:
        c.wait()
    pltpu.sync_copy(rows_v, out_hbm.at[wid])


def kernel(inputs, table):
    idx = inputs.astype(jnp.int32).reshape(NW, NCHUNK, CHUNK)
    out = _sc_gather(idx, table)
    return out.reshape(B, EMBED)


# speedup vs baseline: 1.8984x; 1.8984x over previous
import functools

import jax
import jax.numpy as jnp
from jax import lax
from jax.experimental import pallas as pl
from jax.experimental.pallas import tpu as pltpu
from jax.experimental.pallas import tpu_sc as plsc

EMBED = 64
PADW = 128
B = 16384
VOCAB1 = 1000001
NBLK = (VOCAB1 + PADW - 1) // PADW
NC = 2
NS = 16
NW = NC * NS
BPT = (NBLK + NW - 1) // NW
RING = 4
NGRP = (BPT + RING - 1) // RING
LAST = NBLK - 1
LAST_LO = LAST * PADW

_mesh = plsc.VectorSubcoreMesh(core_axis_name="c", subcore_axis_name="s")


def _ltz(x):
    return lax.shift_right_logical(x, 31)


def _nz(x):
    return lax.shift_right_logical(x | (0 - x), 31)


@functools.partial(
    pl.kernel,
    mesh=_mesh,
    out_type=jax.ShapeDtypeStruct((B, PADW), jnp.float32),
    scratch_types=[
        pltpu.VMEM((B,), jnp.int32),
        pltpu.VMEM((B + 16,), jnp.int32),
        pltpu.VMEM((B + 16,), jnp.int32),
        pltpu.VMEM((RING, EMBED, PADW), jnp.float32),
        pltpu.VMEM((PADW, PADW), jnp.float32),
        pltpu.VMEM((PADW,), jnp.int32),
        pltpu.SMEM((2,), jnp.int32),
        pltpu.SemaphoreType.DMA,
        pltpu.SemaphoreType.DMA,
        pltpu.SemaphoreType.DMA,
        pltpu.SemaphoreType.DMA,
        pltpu.SemaphoreType.DMA,
    ],
    compiler_params=pltpu.CompilerParams(
        use_tc_tiling_on_sc=True, needs_layout_passes=False
    ),
)
def _sc_lookup(idx_hbm, tt_hbm, tail_hbm, out_hbm, idx_all, pairs_i, pairs_r,
               blocks_v, rows_buf, sc_r, cnt_s, sf, s0, s1, s2, s3):
    sems = (s0, s1, s2, s3)
    wid = lax.axis_index("s") * NC + lax.axis_index("c")
    ii = lax.iota(jnp.int32, 16)

    pltpu.sync_copy(idx_hbm, idx_all)

    lo_b = wid * BPT
    hi_b = jnp.minimum(lo_b + BPT, NBLK)
    lo_i = lo_b * PADW
    hi_i = hi_b * PADW

    cnt_s[0] = 0

    def p1(t, c):
        vv = idx_all[pl.ds(t * 16, 16)]
        ge = 1 - _ltz(vv - lo_i)
        lt = _ltz(vv - hi_i)
        mi = ge * lt
        n = cnt_s[0]
        pos = n + plsc.cumsum(mi) - 1
        pos2 = pos * mi + B * (1 - mi)
        plsc.store_scatter(pairs_i, [pos2], vv)
        plsc.store_scatter(pairs_r, [pos2], t * 16 + ii)
        cnt_s[0] = n + jnp.sum(mi)
        return c

    lax.fori_loop(0, B // 16, p1, 0)
    n_pairs = cnt_s[0]
    cnt_s[1] = 0

    def issue(b, u):
        @pl.when(b == LAST)
        def _():
            pltpu.async_copy(tail_hbm, blocks_v.at[u], sems[u])

        @pl.when(b != LAST)
        def _():
            pltpu.async_copy(
                tt_hbm.at[:, pl.ds(pl.multiple_of(b * PADW, PADW), PADW)],
                blocks_v.at[u],
                sems[u],
            )

    def flush_full():
        pltpu.async_copy(rows_buf, out_hbm.at[sc_r], sf).wait()
        cnt_s[1] = 0

    def process(b, u):
        def scan(q, c):
            base = q * 16
            pi = pairs_i[pl.ds(base, 16)]
            pr = pairs_r[pl.ds(base, 16)]
            valid = _ltz((base + ii) - n_pairs)
            mi0 = valid * (1 - _nz(lax.shift_right_logical(pi, 7) ^ b))

            def drain_cond(mi):
                return jnp.sum(mi) > 0

            def drain(mi):
                cs = plsc.cumsum(mi)
                k = jnp.sum(1 - _nz(cs))
                oh = 1 - _nz(ii ^ k)
                col = jnp.sum(oh * (pi & (PADW - 1)))
                row = jnp.sum(oh * pr)
                nloc = cnt_s[1]
                cvec = col + ii * 0
                for q4 in range(EMBED // 16):
                    e = ii + 16 * q4
                    w = plsc.load_gather(blocks_v.at[u], [e, cvec])
                    rows_buf[nloc, pl.ds(16 * q4, 16)] = w
                plsc.store_scatter(sc_r, [nloc + ii * 0], row + ii * 0)
                cnt_s[1] = nloc + 1

                @pl.when(nloc + 1 == PADW)
                def _():
                    flush_full()

                return mi * (1 - oh)

            lax.while_loop(drain_cond, drain, mi0)
            return c

        lax.fori_loop(0, (n_pairs + 15) // 16, scan, 0)

    for u in range(RING):
        b = lo_b + u

        @pl.when(b < hi_b)
        def _(b=b, u=u):
            issue(b, u)

    def grp(g, c):
        for u in range(RING):
            b = lo_b + g * RING + u

            @pl.when(b < hi_b)
            def _(b=b, u=u):
                pltpu.make_async_copy(tail_hbm, blocks_v.at[u], sems[u]).wait()
                process(b, u)
                b2 = b + RING

                @pl.when(b2 < hi_b)
                def _():
                    issue(b2, u)

        return c

    lax.fori_loop(0, NGRP, grp, 0)

    nloc = cnt_s[1]

    @pl.when(nloc > 0)
    def _():
        e0 = 1 - _nz(ii)
        v0 = sc_r[pl.ds(0, 16)]
        r0 = jnp.sum(e0 * v0)
        for j16 in range(PADW // 16):
            pos = ii + 16 * j16
            cur = sc_r[pl.ds(16 * j16, 16)]
            keep = _ltz(pos - nloc)
            sc_r[pl.ds(16 * j16, 16)] = cur * keep + r0 * (1 - keep)
        row0 = [rows_buf[0, pl.ds(16 * q4, 16)] for q4 in range(EMBED // 16)]

        def padrow(j, c):
            @pl.when(j >= nloc)
            def _():
                for q4 in range(EMBED // 16):
                    rows_buf[j, pl.ds(16 * q4, 16)] = row0[q4]

            return c

        lax.fori_loop(0, PADW, padrow, 0)
        flush_full()


def kernel(inputs, table):
    idx = inputs.astype(jnp.int32)
    tt = table.T
    tail = jnp.pad(table[LAST_LO:].T, ((0, 0), (0, PADW - (VOCAB1 - LAST_LO))))
    out = _sc_lookup(idx, tt, tail)
    return out[:, :EMBED]
